# baseline (speedup 1.0000x reference)
import functools

import jax
import jax.numpy as jnp
from jax import lax
from jax.experimental import pallas as pl
from jax.experimental.pallas import tpu as pltpu
from jax.experimental.pallas import tpu_sc as plsc

_N_LAYERS = 26
_VOCAB = 100000
_DIM = 32
_BATCH = 16384

_NUM_CORES = 2
_NUM_SUBCORES = 16
_NW = _NUM_CORES * _NUM_SUBCORES
_RPG = 128
_TOTAL_ROWS = _BATCH * _N_LAYERS
_PER_W = _TOTAL_ROWS // _NW
_STEPS = _PER_W // _RPG
_IDX_ROWS = _TOTAL_ROWS // _RPG
_TROWS = _N_LAYERS * _VOCAB * _DIM // 128
_NSLOT = 2
_NGRP = _STEPS // _NSLOT


def _make_sc_gather():
    mesh = plsc.VectorSubcoreMesh(core_axis_name="c", subcore_axis_name="s")

    @functools.partial(
        pl.kernel,
        mesh=mesh,
        out_type=jax.ShapeDtypeStruct((_TOTAL_ROWS, _DIM), jnp.float32),
        scratch_types=[
            pltpu.VMEM((_STEPS, _RPG), jnp.int32),
            pltpu.VMEM((_STEPS, _RPG), jnp.int32),
            [pltpu.VMEM((_RPG, 128), jnp.float32) for _ in range(_NSLOT)],
            [pltpu.VMEM((_RPG, _DIM), jnp.float32) for _ in range(_NSLOT)],
            [pltpu.SemaphoreType.DMA for _ in range(_NSLOT)],
            [pltpu.SemaphoreType.DMA for _ in range(_NSLOT)],
        ],
    )
    def gather_kernel(gidx_hbm, soff_hbm, table_hbm, out_hbm,
                      gidx_v, soff_v, bigs, rows, gsems, wsems):
        wid = lax.axis_index("s") * _NUM_CORES + lax.axis_index("c")
        idx_row0 = wid * _STEPS
        out_row0 = wid * _PER_W
        pltpu.sync_copy(gidx_hbm.at[pl.ds(idx_row0, _STEPS)], gidx_v)
        pltpu.sync_copy(soff_hbm.at[pl.ds(idx_row0, _STEPS)], soff_v)

        def l1_start(j, s):
            pltpu.async_copy(table_hbm.at[gidx_v.at[j]], bigs[s], gsems[s])

        def l1_wait(s):
            pltpu.make_async_copy(
                table_hbm.at[pl.ds(0, _RPG)], bigs[s], gsems[s]
            ).wait()

        def write_start(j, s):
            pltpu.async_copy(
                rows[s], out_hbm.at[pl.ds(out_row0 + j * _RPG, _RPG)], wsems[s]
            )

        def write_wait(s):
            pltpu.make_async_copy(
                rows[s], out_hbm.at[pl.ds(0, _RPG)], wsems[s]
            ).wait()

        def extract(j, s):
            big = bigs[s]
            row = rows[s]

            def ebody(it, carry):
                base = it * 16
                ovec = soff_v[j, pl.ds(base, 16)]
                for k in range(16):
                    i = base + k
                    off = ovec[k]
                    row[i, pl.ds(0, 16)] = big[i, pl.ds(off, 16)]
                    row[i, pl.ds(16, 16)] = big[i, pl.ds(off + 16, 16)]
                return carry

            lax.fori_loop(0, _RPG // 16, ebody, 0)

        for s in range(_NSLOT):
            l1_start(s, s)

        def body(g, carry):
            for s in range(_NSLOT):
                j = g * _NSLOT + s
                l1_wait(s)

                @pl.when(g > 0)
                def _():
                    write_wait(s)

                extract(j, s)

                @pl.when(g < _NGRP - 1)
                def _():
                    l1_start(j + _NSLOT, s)

                write_start(j, s)
            return carry

        lax.fori_loop(0, _NGRP, body, 0)

        for s in range(_NSLOT):
            write_wait(s)

    return gather_kernel


_sc_gather = _make_sc_gather()


def kernel(x, tables):
    offs = (jnp.arange(_N_LAYERS, dtype=jnp.int32) * _VOCAB)[None, :]
    flat = (x + offs).reshape(-1)
    gidx = (flat >> 2).reshape(_IDX_ROWS, _RPG)
    soff = ((flat & 3) << 5).reshape(_IDX_ROWS, _RPG)
    t128 = tables.reshape(_TROWS, 128)
    out = _sc_gather(gidx, soff, t128)
    return out.reshape(_BATCH, _N_LAYERS * _DIM)
